# baseline (speedup 1.0000x reference)
import functools

import jax
import jax.numpy as jnp
from jax import lax
from jax.experimental import pallas as pl
from jax.experimental.pallas import tpu as pltpu
from jax.experimental.pallas import tpu_sc as plsc

NC, NS, L = 2, 16, 16
NW = NC * NS
BATCH = 16384
NF = 100
NCONT = 20
NCAT = 80
D = 16
TOTAL_TOKENS = 800001
BH = 128
NBH = BATCH // BH
TILES_PER_W = NBH // NW
FB = 10
NBLK = NF // FB


TRB = 1000
TR_PER_W = (TOTAL_TOKENS - 1) // NW


def _transpose_table(table_t):
    mesh = plsc.VectorSubcoreMesh(
        core_axis_name="c", subcore_axis_name="s", num_cores=NC, num_subcores=NS
    )

    @functools.partial(
        pl.kernel,
        out_type=jax.ShapeDtypeStruct((TOTAL_TOKENS, D), jnp.float32),
        mesh=mesh,
        compiler_params=pltpu.CompilerParams(
            needs_layout_passes=False, use_tc_tiling_on_sc=False
        ),
        scratch_types=[
            pltpu.VMEM((D, TRB + 1), jnp.float32),
            pltpu.VMEM((TRB, D), jnp.float32),
        ],
    )
    def tr_kernel(tt_hbm, out_hbm, tin, tout):
        wid = lax.axis_index("s") * NC + lax.axis_index("c")
        r0 = wid * TR_PER_W

        def chunk_body(c, carry):
            r = r0 + c * TRB
            pltpu.sync_copy(tt_hbm.at[:, pl.ds(r, TRB)], tin.at[:, pl.ds(0, TRB)])

            @plsc.parallel_loop(0, TRB, unroll=1)
            def tr_body(i):
                tout[i, :] = plsc.load_gather(
                    tin, [lax.iota(jnp.int32, L), jnp.full((L,), i, jnp.int32)]
                )

            pltpu.sync_copy(tout, out_hbm.at[pl.ds(r, TRB)])
            return carry

        lax.fori_loop(0, TR_PER_W // TRB, chunk_body, 0)

        @pl.when(wid == 0)
        def _tail():
            pltpu.sync_copy(
                tt_hbm.at[:, pl.ds(TOTAL_TOKENS - 1, 1)], tin.at[:, pl.ds(0, 1)]
            )
            tout[0, :] = plsc.load_gather(
                tin, [lax.iota(jnp.int32, L), jnp.full((L,), 0, jnp.int32)]
            )
            pltpu.sync_copy(
                tout.at[pl.ds(0, 1)], out_hbm.at[pl.ds(TOTAL_TOKENS - 1, 1)]
            )

    return tr_kernel(table_t)


def kernel(inputs, categorical_embeddings, continuous_embeddings, bias):
    mesh = plsc.VectorSubcoreMesh(
        core_axis_name="c", subcore_axis_name="s", num_cores=NC, num_subcores=NS
    )

    @functools.partial(
        pl.kernel,
        out_type=jax.ShapeDtypeStruct((NF * 2 * NBH * 8 * BH,), jnp.float32),
        mesh=mesh,
        compiler_params=pltpu.CompilerParams(
            needs_layout_passes=False, use_tc_tiling_on_sc=False
        ),
        scratch_types=[
            pltpu.VMEM((NF, BH), jnp.float32),
            pltpu.VMEM((NCAT, BH), jnp.int32),
            pltpu.VMEM((2, FB * BH, D), jnp.float32),
            pltpu.VMEM((2, FB * D * BH), jnp.float32),
            pltpu.VMEM((NF, D), jnp.float32),
            pltpu.VMEM((NCONT * D, D), jnp.float32),
            pltpu.VMEM((FB, L), jnp.int32),
            pltpu.VMEM((L + NCONT * D,), jnp.float32),
            pltpu.VMEM((L + NF,), jnp.float32),
            pltpu.SemaphoreType.DMA,
            pltpu.SemaphoreType.DMA,
            pltpu.SemaphoreType.DMA,
        ],
    )
    def sc_kernel(xt_hbm, table_hbm, cont_hbm, bias_hbm, out_hbm,
                  xchunk, idxb, gbuf, stag, bb, cesp, sbase, ce, bv,
                  gsem, osem0, osem1):
        osems = (osem0, osem1)
        wid = lax.axis_index("s") * NC + lax.axis_index("c")

        pltpu.sync_copy(cont_hbm, ce.at[pl.ds(L, NCONT * D)])
        pltpu.sync_copy(bias_hbm, bv.at[pl.ds(L, NF)])
        for f in range(NF):
            bb[f, :] = plsc.load_gather(bv, [jnp.full((L,), L + f, jnp.int32)])
        for f in range(NCONT):
            for d in range(D):
                cesp[f * D + d, :] = plsc.load_gather(
                    ce, [jnp.full((L,), L + f * D + d, jnp.int32)]
                )
        for fl in range(FB):
            sbase[fl, :] = (fl * D + lax.iota(jnp.int32, L)) * BH

        def fire_gathers(blk):
            p = blk & 1
            for fl in range(FB):
                fc = blk * FB - NCONT + fl
                pltpu.async_copy(
                    table_hbm.at[idxb.at[fc]],
                    gbuf.at[p, pl.ds(fl * BH, BH)], gsem,
                )

        def wait_gathers(blk):
            p = blk & 1
            for fl in range(FB):
                fc = blk * FB - NCONT + fl
                pltpu.make_async_copy(
                    table_hbm.at[idxb.at[fc]],
                    gbuf.at[p, pl.ds(fl * BH, BH)], gsem,
                ).wait()

        def drain_out(p):
            for _ in range(2 * FB):
                pltpu.make_async_copy(
                    stag.at[0, pl.ds(0, 8 * BH)],
                    out_hbm.at[pl.ds(0, 8 * BH)], osems[p],
                ).wait()

        def fire_out(blk, bh):
            p = blk & 1
            for fl in range(FB):
                fo = blk * FB + fl
                for dh in range(2):
                    r = ((fo * 2 + dh) * NBH + bh) * 8 * BH
                    pltpu.async_copy(
                        stag.at[p, pl.ds((fl * D + dh * 8) * BH, 8 * BH)],
                        out_hbm.at[pl.ds(r, 8 * BH)], osems[p],
                    )

        def tile_body(c, carry):
            bh = wid * TILES_PER_W + c
            b0 = bh * BH
            pltpu.sync_copy(xt_hbm.at[:, pl.ds(b0, BH)], xchunk)

            @plsc.parallel_loop(0, NCAT, unroll=1)
            def idx_body(f):
                offs = jnp.full((L,), 1, jnp.int32) + f * 10000
                for k in range(BH // L):
                    xv = xchunk[NCONT + f, pl.ds(L * k, L)]
                    idxb[f, pl.ds(L * k, L)] = xv.astype(jnp.int32) + offs

            fire_gathers(2)
            for blk in range(NBLK):
                p = blk & 1
                if 3 <= blk + 1 < NBLK:
                    fire_gathers(blk + 1)
                if blk >= 2:
                    wait_gathers(blk)
                if blk >= 2:
                    drain_out(p)
                if blk < 2:
                    def cont_body(fl, _):
                        f = blk * FB + fl
                        biasv = bb[f, :]
                        for k in range(BH // L):
                            xv = xchunk[f, pl.ds(L * k, L)]
                            for d in range(D):
                                ev = cesp[f * D + d, :]
                                stag[p, pl.ds((fl * D + d) * BH + L * k, L)] = (
                                    xv * ev + biasv
                                )
                        return _
                    lax.fori_loop(0, FB, cont_body, 0)
                else:
                    biases = [bb[blk * FB + fl, :] for fl in range(FB)]
                    sbs = [sbase[fl, :] for fl in range(FB)]
                    @plsc.parallel_loop(0, BH, unroll=1)
                    def sc_body(b):
                        bvv = jnp.full((L,), b, jnp.int32)
                        for fl in range(FB):
                            row = gbuf[p, fl * BH + b, :] + biases[fl]
                            plsc.store_scatter(
                                stag.at[p], [sbs[fl] + bvv], row
                            )
                fire_out(blk, bh)
            drain_out(0)
            drain_out(1)
            return carry

        lax.fori_loop(0, TILES_PER_W, tile_body, 0)

    table_rm = _transpose_table(categorical_embeddings.T)
    out = sc_kernel(
        inputs.T, table_rm, continuous_embeddings.reshape(-1), bias
    )
    out = out.reshape(NF, 2, NBH, 8, BH).transpose(2, 4, 0, 1, 3)
    return out.reshape(BATCH, NF, D)


# speedup vs baseline: 2.2624x; 2.2624x over previous
import functools

import jax
import jax.numpy as jnp
from jax import lax
from jax.experimental import pallas as pl
from jax.experimental.pallas import tpu as pltpu
from jax.experimental.pallas import tpu_sc as plsc

NC, NS, L = 2, 16, 16
NW = NC * NS
BATCH = 16384
NF = 100
NCONT = 20
NCAT = 80
D = 16
TOTAL_TOKENS = 800001
BH = 128
NBH = BATCH // BH
TILES_PER_W = NBH // NW
FB = 10
NBLK = NF // FB


def kernel(inputs, categorical_embeddings, continuous_embeddings, bias):
    mesh = plsc.VectorSubcoreMesh(
        core_axis_name="c", subcore_axis_name="s", num_cores=NC, num_subcores=NS
    )

    @functools.partial(
        pl.kernel,
        out_type=jax.ShapeDtypeStruct((NF * 2 * NBH * 8 * BH,), jnp.float32),
        mesh=mesh,
        compiler_params=pltpu.CompilerParams(
            needs_layout_passes=False, use_tc_tiling_on_sc=False
        ),
        scratch_types=[
            pltpu.VMEM((NF, BH), jnp.float32),
            pltpu.VMEM((NCAT, BH), jnp.int32),
            pltpu.VMEM((2, FB * BH, D), jnp.float32),
            pltpu.VMEM((2, FB * D * BH), jnp.float32),
            pltpu.VMEM((NF, D), jnp.float32),
            pltpu.VMEM((NCONT * D, D), jnp.float32),
            pltpu.VMEM((FB, L), jnp.int32),
            pltpu.VMEM((L + NCONT * D,), jnp.float32),
            pltpu.VMEM((L + NF,), jnp.float32),
            pltpu.SemaphoreType.DMA,
            pltpu.SemaphoreType.DMA,
            pltpu.SemaphoreType.DMA,
        ],
    )
    def sc_kernel(xt_hbm, table_hbm, cont_hbm, bias_hbm, out_hbm,
                  xchunk, idxb, gbuf, stag, bb, cesp, sbase, ce, bv,
                  gsem, osem0, osem1):
        osems = (osem0, osem1)
        wid = lax.axis_index("s") * NC + lax.axis_index("c")

        pltpu.sync_copy(cont_hbm, ce.at[pl.ds(L, NCONT * D)])
        pltpu.sync_copy(bias_hbm, bv.at[pl.ds(L, NF)])
        for f in range(NF):
            bb[f, :] = plsc.load_gather(bv, [jnp.full((L,), L + f, jnp.int32)])
        for f in range(NCONT):
            for d in range(D):
                cesp[f * D + d, :] = plsc.load_gather(
                    ce, [jnp.full((L,), L + f * D + d, jnp.int32)]
                )
        for fl in range(FB):
            sbase[fl, :] = (fl * D + lax.iota(jnp.int32, L)) * BH

        def fire_gathers(blk):
            p = blk & 1
            for fl in range(FB):
                fc = blk * FB - NCONT + fl
                pltpu.async_copy(
                    table_hbm.at[idxb.at[fc]],
                    gbuf.at[p, pl.ds(fl * BH, BH)], gsem,
                )

        def wait_gathers(blk):
            p = blk & 1
            for fl in range(FB):
                fc = blk * FB - NCONT + fl
                pltpu.make_async_copy(
                    table_hbm.at[idxb.at[fc]],
                    gbuf.at[p, pl.ds(fl * BH, BH)], gsem,
                ).wait()

        def drain_out(p):
            for _ in range(2 * FB):
                pltpu.make_async_copy(
                    stag.at[0, pl.ds(0, 8 * BH)],
                    out_hbm.at[pl.ds(0, 8 * BH)], osems[p],
                ).wait()

        def fire_out(blk, bh):
            p = blk & 1
            for fl in range(FB):
                fo = blk * FB + fl
                for dh in range(2):
                    r = ((fo * 2 + dh) * NBH + bh) * 8 * BH
                    pltpu.async_copy(
                        stag.at[p, pl.ds((fl * D + dh * 8) * BH, 8 * BH)],
                        out_hbm.at[pl.ds(r, 8 * BH)], osems[p],
                    )

        def tile_body(c, carry):
            bh = wid * TILES_PER_W + c
            b0 = bh * BH
            pltpu.sync_copy(xt_hbm.at[:, pl.ds(b0, BH)], xchunk)

            @plsc.parallel_loop(0, NCAT, unroll=1)
            def idx_body(f):
                offs = jnp.full((L,), 1, jnp.int32) + f * 10000
                for k in range(BH // L):
                    xv = xchunk[NCONT + f, pl.ds(L * k, L)]
                    idxb[f, pl.ds(L * k, L)] = xv.astype(jnp.int32) + offs

            fire_gathers(2)
            for blk in range(NBLK):
                p = blk & 1
                if 3 <= blk + 1 < NBLK:
                    fire_gathers(blk + 1)
                if blk >= 2:
                    wait_gathers(blk)
                if blk >= 2:
                    drain_out(p)
                if blk < 2:
                    def cont_body(fl, _):
                        f = blk * FB + fl
                        biasv = bb[f, :]
                        for k in range(BH // L):
                            xv = xchunk[f, pl.ds(L * k, L)]
                            for d in range(D):
                                ev = cesp[f * D + d, :]
                                stag[p, pl.ds((fl * D + d) * BH + L * k, L)] = (
                                    xv * ev + biasv
                                )
                        return _
                    lax.fori_loop(0, FB, cont_body, 0)
                else:
                    biases = [bb[blk * FB + fl, :] for fl in range(FB)]
                    sbs = [sbase[fl, :] for fl in range(FB)]
                    @plsc.parallel_loop(0, BH, unroll=2)
                    def sc_body(b):
                        bvv = jnp.full((L,), b, jnp.int32)
                        for fl in range(FB):
                            row = gbuf[p, fl * BH + b, :] + biases[fl]
                            plsc.store_scatter(
                                stag.at[p], [sbs[fl] + bvv], row
                            )
                fire_out(blk, bh)
            drain_out(0)
            drain_out(1)
            return carry

        lax.fori_loop(0, TILES_PER_W, tile_body, 0)

    out = sc_kernel(
        inputs.T, categorical_embeddings, continuous_embeddings.reshape(-1), bias
    )
    out = out.reshape(NF, 2, NBH, 8, BH).transpose(2, 4, 0, 1, 3)
    return out.reshape(BATCH, NF, D)
